# baseline (speedup 1.0000x reference)
import functools

import jax
import jax.numpy as jnp
from jax import lax
from jax.experimental import pallas as pl
from jax.experimental.pallas import tpu as pltpu
from jax.experimental.pallas import tpu_sc as plsc

_B = 4096
_D = 64
_V = 100000

_TN = 1024
_NV = pl.cdiv(_V, _TN)
_TLAST = _V - (_NV - 1) * _TN
_NBUF = 3


def _make_sc_gather():
    info = plsc.get_sparse_core_info()
    nw = info.num_cores * info.num_subcores
    b_per_w = _B // nw
    mesh = plsc.VectorSubcoreMesh(core_axis_name="c", subcore_axis_name="s")

    @functools.partial(
        pl.kernel,
        mesh=mesh,
        out_type=jax.ShapeDtypeStruct((_B, _D), jnp.float32),
        scratch_types=[
            pltpu.VMEM((b_per_w,), jnp.int32),
            pltpu.VMEM((b_per_w, _D), jnp.float32),
            pltpu.SemaphoreType.DMA,
        ],
        compiler_params=pltpu.CompilerParams(use_tc_tiling_on_sc=False),
    )
    def gather_kernel(table_hbm, idx_hbm, out_hbm, idx_v, rows_v, sem):
        wid = lax.axis_index("s") * info.num_cores + lax.axis_index("c")
        base = wid * b_per_w
        pltpu.sync_copy(idx_hbm.at[pl.ds(base, b_per_w)], idx_v)
        pltpu.async_copy(table_hbm.at[idx_v], rows_v, sem).wait()
        pltpu.sync_copy(rows_v, out_hbm.at[pl.ds(base, b_per_w)])

    return gather_kernel


def _ocopy(vv, slot, o_hbm, acc, sems):
    rows = _TLAST if vv == _NV - 1 else _TN
    return pltpu.make_async_copy(
        acc.at[slot, pl.ds(0, rows)],
        o_hbm.at[pl.ds(vv * _TN, rows)],
        sems.at[slot],
    )


def _mm_body(h_ref, w2t_ref, o_hbm, acc, sems):
    v = pl.program_id(0)
    slot = lax.rem(v, _NBUF)

    @pl.when(v >= _NBUF)
    def _wait_prev():
        pltpu.make_async_copy(
            acc.at[slot], o_hbm.at[pl.ds((v - _NBUF) * _TN, _TN)], sems.at[slot]
        ).wait()

    acc[slot] = lax.dot_general(
        w2t_ref[...],
        h_ref[...],
        (((0,), (1,)), ((), ())),
        preferred_element_type=jnp.float32,
    )

    @pl.when(v < _NV - 1)
    def _start_full():
        pltpu.make_async_copy(
            acc.at[slot], o_hbm.at[pl.ds(v * _TN, _TN)], sems.at[slot]
        ).start()

    @pl.when(v == _NV - 1)
    def _last():
        _ocopy(_NV - 1, slot, o_hbm, acc, sems).start()
        for k in range(_NBUF):
            vv = _NV - _NBUF + k
            _ocopy(vv, lax.rem(vv, _NBUF), o_hbm, acc, sems).wait()


def _projection_t(hidden, W2t):
    nv = _NV
    return pl.pallas_call(
        _mm_body,
        grid=(nv,),
        in_specs=[
            pl.BlockSpec((_B, _D), lambda v: (0, 0)),
            pl.BlockSpec((_D, _TN), lambda v: (0, v)),
        ],
        out_specs=pl.BlockSpec(memory_space=pl.ANY),
        out_shape=jax.ShapeDtypeStruct((_V, _B), jnp.float32),
        scratch_shapes=[
            pltpu.VMEM((_NBUF, _TN, _B), jnp.float32),
            pltpu.SemaphoreType.DMA((_NBUF,)),
        ],
        compiler_params=pltpu.CompilerParams(
            vmem_limit_bytes=110 * 1024 * 1024,
        ),
    )(hidden, W2t)


_sc_gather = _make_sc_gather()


@jax.jit
def kernel(X, W, W2):
    hidden = _sc_gather(W, X.astype(jnp.int32))
    return _projection_t(hidden, W2.T).T


# speedup vs baseline: 1.0001x; 1.0001x over previous
import functools

import jax
import jax.numpy as jnp
from jax import lax
from jax.experimental import pallas as pl
from jax.experimental.pallas import tpu as pltpu
from jax.experimental.pallas import tpu_sc as plsc

_B = 4096
_D = 64
_V = 100000

_TN = 1024
_NV = pl.cdiv(_V, _TN)
_TLAST = _V - (_NV - 1) * _TN
_NBUF = 3


def _make_sc_gather():
    info = plsc.get_sparse_core_info()
    nw = info.num_cores * info.num_subcores
    b_per_w = _B // nw
    mesh = plsc.VectorSubcoreMesh(core_axis_name="c", subcore_axis_name="s")

    @functools.partial(
        pl.kernel,
        mesh=mesh,
        out_type=jax.ShapeDtypeStruct((_B, _D), jnp.float32),
        scratch_types=[
            pltpu.VMEM((b_per_w,), jnp.int32),
            pltpu.VMEM((b_per_w, _D), jnp.float32),
            pltpu.SemaphoreType.DMA,
        ],
        compiler_params=pltpu.CompilerParams(use_tc_tiling_on_sc=False),
    )
    def gather_kernel(table_hbm, idx_hbm, out_hbm, idx_v, rows_v, sem):
        wid = lax.axis_index("s") * info.num_cores + lax.axis_index("c")
        base = wid * b_per_w
        pltpu.sync_copy(idx_hbm.at[pl.ds(base, b_per_w)], idx_v)
        pltpu.async_copy(table_hbm.at[idx_v], rows_v, sem).wait()
        pltpu.sync_copy(rows_v, out_hbm.at[pl.ds(base, b_per_w)])

    return gather_kernel


_S = 4
_TS = _TN // _S


def _stripes(vv):
    total = _TLAST if vv == _NV - 1 else _TN
    return [
        (s * _TS, max(0, min(_TS, total - s * _TS))) for s in range(_S)
    ]


def _odma(vv, vv_off, slot, o_hbm, acc, sems, start):
    for s, (lo, rows) in enumerate(_stripes(vv)):
        if rows:
            cp = pltpu.make_async_copy(
                acc.at[slot, pl.ds(lo, rows)],
                o_hbm.at[pl.ds(vv_off + lo, rows)],
                sems.at[slot, s],
            )
            cp.start() if start else cp.wait()


def _mm_body(h_ref, w2t_ref, o_hbm, acc, sems):
    v = pl.program_id(0)
    slot = lax.rem(v, _NBUF)

    @pl.when(v >= _NBUF)
    def _wait_prev():
        _odma(0, (v - _NBUF) * _TN, slot, o_hbm, acc, sems, start=False)

    acc[slot] = lax.dot_general(
        w2t_ref[...],
        h_ref[...],
        (((0,), (1,)), ((), ())),
        preferred_element_type=jnp.float32,
    )

    @pl.when(v < _NV - 1)
    def _start_full():
        _odma(0, v * _TN, slot, o_hbm, acc, sems, start=True)

    @pl.when(v == _NV - 1)
    def _last():
        _odma(_NV - 1, v * _TN, slot, o_hbm, acc, sems, start=True)
        for k in range(_NBUF):
            vv = _NV - _NBUF + k
            _odma(vv, vv * _TN, lax.rem(vv, _NBUF), o_hbm, acc, sems, start=False)


def _projection_t(hidden, W2t):
    nv = _NV
    return pl.pallas_call(
        _mm_body,
        grid=(nv,),
        in_specs=[
            pl.BlockSpec((_B, _D), lambda v: (0, 0)),
            pl.BlockSpec((_D, _TN), lambda v: (0, v)),
        ],
        out_specs=pl.BlockSpec(memory_space=pl.ANY),
        out_shape=jax.ShapeDtypeStruct((_V, _B), jnp.float32),
        scratch_shapes=[
            pltpu.VMEM((_NBUF, _TN, _B), jnp.float32),
            pltpu.SemaphoreType.DMA((_NBUF, _S)),
        ],
        compiler_params=pltpu.CompilerParams(
            vmem_limit_bytes=110 * 1024 * 1024,
        ),
    )(hidden, W2t)


_sc_gather = _make_sc_gather()


@jax.jit
def kernel(X, W, W2):
    hidden = _sc_gather(W, X.astype(jnp.int32))
    return _projection_t(hidden, W2.T).T


# speedup vs baseline: 1.0308x; 1.0307x over previous
import functools

import jax
import jax.numpy as jnp
from jax import lax
from jax.experimental import pallas as pl
from jax.experimental.pallas import tpu as pltpu
from jax.experimental.pallas import tpu_sc as plsc

_B = 4096
_D = 64
_V = 100000

_TN = 1024
_NV = pl.cdiv(_V, _TN)
_TLAST = _V - (_NV - 1) * _TN
_NBUF = 3


def _make_sc_gather():
    info = plsc.get_sparse_core_info()
    nw = info.num_cores * info.num_subcores
    b_per_w = _B // nw
    mesh = plsc.VectorSubcoreMesh(core_axis_name="c", subcore_axis_name="s")

    @functools.partial(
        pl.kernel,
        mesh=mesh,
        out_type=jax.ShapeDtypeStruct((_D, _B), jnp.float32),
        scratch_types=[
            pltpu.VMEM((b_per_w,), jnp.int32),
            pltpu.VMEM((_D, b_per_w), jnp.int32),
            pltpu.VMEM((_D, b_per_w), jnp.float32),
            pltpu.SemaphoreType.DMA,
        ],
        compiler_params=pltpu.CompilerParams(use_tc_tiling_on_sc=False),
    )
    def gather_kernel(table_hbm, idx_hbm, out_hbm, idx_v, idx2, rows_v, sem):
        wid = lax.axis_index("s") * info.num_cores + lax.axis_index("c")
        base = wid * b_per_w
        pltpu.sync_copy(idx_hbm.at[pl.ds(base, b_per_w)], idx_v)

        def build(k, c):
            for cc in range(b_per_w // 16):
                sl = pl.ds(cc * 16, 16)
                idx2[k, sl] = idx_v[sl] + k * _V
            return c

        lax.fori_loop(0, _D, build, 0)

        def fire(k, c):
            pltpu.async_copy(table_hbm.at[idx2.at[k]], rows_v.at[k], sem)
            return c

        lax.fori_loop(0, _D, fire, 0)

        def drain(k, c):
            pltpu.make_async_copy(
                table_hbm.at[idx2.at[k]], rows_v.at[k], sem
            ).wait()
            return c

        lax.fori_loop(0, _D, drain, 0)
        pltpu.sync_copy(rows_v, out_hbm.at[:, pl.ds(base, b_per_w)])

    return gather_kernel


_S = 4
_TS = _TN // _S


def _stripes(vv):
    total = _TLAST if vv == _NV - 1 else _TN
    return [
        (s * _TS, max(0, min(_TS, total - s * _TS))) for s in range(_S)
    ]


def _odma(vv, vv_off, slot, o_hbm, acc, sems, start):
    for s, (lo, rows) in enumerate(_stripes(vv)):
        if rows:
            cp = pltpu.make_async_copy(
                acc.at[slot, pl.ds(lo, rows)],
                o_hbm.at[pl.ds(vv_off + lo, rows)],
                sems.at[slot, s],
            )
            cp.start() if start else cp.wait()


def _mm_body(h_ref, w2t_ref, o_hbm, acc, sems):
    v = pl.program_id(0)
    slot = lax.rem(v, _NBUF)

    @pl.when(v >= _NBUF)
    def _wait_prev():
        _odma(0, (v - _NBUF) * _TN, slot, o_hbm, acc, sems, start=False)

    acc[slot] = lax.dot_general(
        w2t_ref[...],
        h_ref[...],
        (((0,), (0,)), ((), ())),
        preferred_element_type=jnp.float32,
    )

    @pl.when(v < _NV - 1)
    def _start_full():
        _odma(0, v * _TN, slot, o_hbm, acc, sems, start=True)

    @pl.when(v == _NV - 1)
    def _last():
        _odma(_NV - 1, v * _TN, slot, o_hbm, acc, sems, start=True)
        for k in range(_NBUF):
            vv = _NV - _NBUF + k
            _odma(vv, vv * _TN, lax.rem(vv, _NBUF), o_hbm, acc, sems, start=False)


def _projection_t(hidden, W2t):
    nv = _NV
    return pl.pallas_call(
        _mm_body,
        grid=(nv,),
        in_specs=[
            pl.BlockSpec((_D, _B), lambda v: (0, 0)),
            pl.BlockSpec((_D, _TN), lambda v: (0, v)),
        ],
        out_specs=pl.BlockSpec(memory_space=pl.ANY),
        out_shape=jax.ShapeDtypeStruct((_V, _B), jnp.float32),
        scratch_shapes=[
            pltpu.VMEM((_NBUF, _TN, _B), jnp.float32),
            pltpu.SemaphoreType.DMA((_NBUF, _S)),
        ],
        compiler_params=pltpu.CompilerParams(
            vmem_limit_bytes=110 * 1024 * 1024,
        ),
    )(hidden, W2t)


_sc_gather = _make_sc_gather()


@jax.jit
def kernel(X, W, W2):
    hidden_t = _sc_gather(W.T.reshape(-1), X.astype(jnp.int32))
    return _projection_t(hidden_t, W2.T).T
